# baseline (speedup 1.0000x reference)
import functools

import jax
import jax.numpy as jnp
from jax import lax
from jax.experimental import pallas as pl
from jax.experimental.pallas import tpu as pltpu
from jax.experimental.pallas import tpu_sc as plsc

VOCAB = 1000000
DIM = 64
BATCH = 16384
HIST = 50

_NW = 32
_BPW = BATCH // _NW
_BT = 4
_NU = _BT * HIST


def _body(xt_hbm, table_hbm, out_hbm, idx_ht, g0, g1, t0, t1, gs0, gs1, ws0, ws1):
    wid = lax.axis_index("s") * 2 + lax.axis_index("c")
    b0 = wid * _BPW
    pltpu.sync_copy(xt_hbm.at[:, pl.ds(b0, _BPW)], idx_ht)

    gbufs = (g0, g1)
    tbufs = (t0, t1)
    gsems = (gs0, gs1)
    wsems = (ws0, ws1)
    iota = lax.iota(jnp.int32, 16)

    def unit_hbt(u):
        return u // _BT, u % _BT

    def fire(u, p):
        h, btl = unit_hbt(u)
        pltpu.async_copy(
            table_hbm.at[idx_ht.at[h, pl.ds(btl * 128, 128)]],
            gbufs[p],
            gsems[p],
        )

    def drain_gather(p):
        pltpu.make_async_copy(
            table_hbm.at[idx_ht.at[0, pl.ds(0, 128)]], gbufs[p], gsems[p]
        ).wait()

    def drain_write(p):
        pltpu.make_async_copy(
            tbufs[p], out_hbm.at[0, :, 0], wsems[p]
        ).wait()

    fire(0, 0)

    @pl.loop(0, _NU, step=2)
    def _(u):
        for p in range(2):
            uu = u + p
            h, btl = unit_hbt(uu)

            @pl.when(uu + 1 < _NU)
            def _():
                fire(uu + 1, 1 - p)

            drain_gather(p)

            @pl.when(uu >= 2)
            def _():
                drain_write(p)

            for dt in range(8):
                for dr in range(8):
                    d = dt * 8 + dr
                    for brg in range(8):
                        rows = iota + brg * 16
                        cols = lax.broadcast(jnp.int32(d), (16,))
                        v = plsc.load_gather(gbufs[p], [rows, cols])
                        tbufs[p][dt, dr, pl.ds(brg * 16, 16)] = v

            pltpu.async_copy(
                tbufs[p], out_hbm.at[h, :, wid * _BT + btl], wsems[p]
            )

    drain_write(0)
    drain_write(1)


@jax.jit
def kernel(x, table):
    mesh = plsc.VectorSubcoreMesh(core_axis_name="c", subcore_axis_name="s")
    out5 = pl.kernel(
        _body,
        out_type=jax.ShapeDtypeStruct((HIST, 8, 128, 8, 128), jnp.float32),
        mesh=mesh,
        scratch_types=[
            pltpu.VMEM((HIST, _BPW), jnp.int32),
            pltpu.VMEM((128, DIM), jnp.float32),
            pltpu.VMEM((128, DIM), jnp.float32),
            pltpu.VMEM((8, 8, 128), jnp.float32),
            pltpu.VMEM((8, 8, 128), jnp.float32),
            pltpu.SemaphoreType.DMA,
            pltpu.SemaphoreType.DMA,
            pltpu.SemaphoreType.DMA,
            pltpu.SemaphoreType.DMA,
        ],
        compiler_params=pltpu.CompilerParams(
            use_tc_tiling_on_sc=False, needs_layout_passes=False
        ),
    )(x.T.astype(jnp.int32), table)
    return out5.transpose(2, 4, 0, 1, 3).reshape(BATCH, HIST, DIM)


# speedup vs baseline: 1.0000x; 1.0000x over previous
import functools

import jax
import jax.numpy as jnp
from jax import lax
from jax.experimental import pallas as pl
from jax.experimental.pallas import tpu as pltpu
from jax.experimental.pallas import tpu_sc as plsc

VOCAB = 1000000
DIM = 64
BATCH = 16384
HIST = 50

_NW = 32
_BPW = BATCH // _NW
_BT = 4
_NU = _BT * HIST


def _body(xt_hbm, table_hbm, out_hbm, idx_ht, g0, g1, t0, t1, gs0, gs1, ws0, ws1):
    wid = lax.axis_index("s") * 2 + lax.axis_index("c")
    b0 = wid * _BPW
    pltpu.sync_copy(xt_hbm.at[:, pl.ds(b0, _BPW)], idx_ht)

    gbufs = (g0, g1)
    tbufs = (t0, t1)
    gsems = (gs0, gs1)
    wsems = (ws0, ws1)
    iota = lax.iota(jnp.int32, 16)
    zero16 = iota * 0
    rows16 = [iota + brg * 16 for brg in range(8)]

    def unit_hbt(u):
        return u // _BT, u % _BT

    def fire(u, p):
        h, btl = unit_hbt(u)
        pltpu.async_copy(
            table_hbm.at[idx_ht.at[h, pl.ds(btl * 128, 128)]],
            gbufs[p],
            gsems[p],
        )

    def drain_gather(p):
        pltpu.make_async_copy(
            table_hbm.at[idx_ht.at[0, pl.ds(0, 128)]], gbufs[p], gsems[p]
        ).wait()

    def drain_write(p):
        pltpu.make_async_copy(
            tbufs[p], out_hbm.at[0, :, 0], wsems[p]
        ).wait()

    fire(0, 0)

    @pl.loop(0, _NU, step=2)
    def _(u):
        for p in range(2):
            uu = u + p
            h, btl = unit_hbt(uu)

            @pl.when(uu + 1 < _NU)
            def _():
                fire(uu + 1, 1 - p)

            drain_gather(p)

            @pl.when(uu >= 2)
            def _():
                drain_write(p)

            for d in range(DIM):
                col = zero16 + d
                for brg in range(8):
                    v = plsc.load_gather(gbufs[p], [rows16[brg], col])
                    tbufs[p][d // 8, d % 8, pl.ds(brg * 16, 16)] = v

            pltpu.async_copy(
                tbufs[p], out_hbm.at[h, :, wid * _BT + btl], wsems[p]
            )

    drain_write(0)
    drain_write(1)


@jax.jit
def kernel(x, table):
    mesh = plsc.VectorSubcoreMesh(core_axis_name="c", subcore_axis_name="s")
    out5 = pl.kernel(
        _body,
        out_type=jax.ShapeDtypeStruct((HIST, 8, 128, 8, 128), jnp.float32),
        mesh=mesh,
        scratch_types=[
            pltpu.VMEM((HIST, _BPW), jnp.int32),
            pltpu.VMEM((128, DIM), jnp.float32),
            pltpu.VMEM((128, DIM), jnp.float32),
            pltpu.VMEM((8, 8, 128), jnp.float32),
            pltpu.VMEM((8, 8, 128), jnp.float32),
            pltpu.SemaphoreType.DMA,
            pltpu.SemaphoreType.DMA,
            pltpu.SemaphoreType.DMA,
            pltpu.SemaphoreType.DMA,
        ],
        compiler_params=pltpu.CompilerParams(
            use_tc_tiling_on_sc=False, needs_layout_passes=False
        ),
    )(x.T.astype(jnp.int32), table)
    return out5.transpose(2, 4, 0, 1, 3).reshape(BATCH, HIST, DIM)


# speedup vs baseline: 1.0246x; 1.0245x over previous
import functools

import jax
import jax.numpy as jnp
from jax import lax
from jax.experimental import pallas as pl
from jax.experimental.pallas import tpu as pltpu
from jax.experimental.pallas import tpu_sc as plsc

VOCAB = 1000000
DIM = 64
BATCH = 16384
HIST = 50

_NW = 32
_BPW = BATCH // _NW
_BT = 4
_NU = _BT * HIST


def _body(xt_hbm, table_hbm, out_hbm, idx_ht,
          g0, g1, g2, g3, t0, t1, t2, t3,
          gs0, gs1, gs2, gs3, ws0, ws1, ws2, ws3):
    wid = lax.axis_index("s") * 2 + lax.axis_index("c")
    b0 = wid * _BPW
    pltpu.sync_copy(xt_hbm.at[:, pl.ds(b0, _BPW)], idx_ht)

    gbufs = (g0, g1, g2, g3)
    tbufs = (t0, t1, t2, t3)
    gsems = (gs0, gs1, gs2, gs3)
    wsems = (ws0, ws1, ws2, ws3)
    iota = lax.iota(jnp.int32, 16)
    zero16 = iota * 0
    rows16 = [iota + brg * 16 for brg in range(8)]

    def unit_hbt(u):
        return u // _BT, u % _BT

    def fire(u, p):
        h, btl = unit_hbt(u)
        pltpu.async_copy(
            table_hbm.at[idx_ht.at[h, pl.ds(btl * 128, 128)]],
            gbufs[p],
            gsems[p],
        )

    def drain_gather(p):
        pltpu.make_async_copy(
            table_hbm.at[idx_ht.at[0, pl.ds(0, 128)]], gbufs[p], gsems[p]
        ).wait()

    def drain_write(p):
        pltpu.make_async_copy(
            tbufs[p], out_hbm.at[0, :, 0], wsems[p]
        ).wait()

    fire(0, 0)
    fire(1, 1)
    fire(2, 2)

    @pl.loop(0, _NU, step=4)
    def _(u):
        for p in range(4):
            uu = u + p
            h, btl = unit_hbt(uu)

            @pl.when(uu + 3 < _NU)
            def _():
                fire(uu + 3, (p + 3) % 4)

            drain_gather(p)

            @pl.when(uu >= 4)
            def _():
                drain_write(p)

            for d in range(DIM):
                col = zero16 + d
                for brg in range(8):
                    v = plsc.load_gather(gbufs[p], [rows16[brg], col])
                    tbufs[p][d // 8, d % 8, pl.ds(brg * 16, 16)] = v

            pltpu.async_copy(
                tbufs[p], out_hbm.at[h, :, wid * _BT + btl], wsems[p]
            )

    drain_write(0)
    drain_write(1)
    drain_write(2)
    drain_write(3)


@jax.jit
def kernel(x, table):
    mesh = plsc.VectorSubcoreMesh(core_axis_name="c", subcore_axis_name="s")
    out5 = pl.kernel(
        _body,
        out_type=jax.ShapeDtypeStruct((HIST, 8, 128, 8, 128), jnp.float32),
        mesh=mesh,
        scratch_types=(
            [pltpu.VMEM((HIST, _BPW), jnp.int32)]
            + [pltpu.VMEM((128, DIM), jnp.float32)] * 4
            + [pltpu.VMEM((8, 8, 128), jnp.float32)] * 4
            + [pltpu.SemaphoreType.DMA] * 8
        ),
        compiler_params=pltpu.CompilerParams(
            use_tc_tiling_on_sc=False, needs_layout_passes=False
        ),
    )(x.T.astype(jnp.int32), table)
    return out5.transpose(2, 4, 0, 1, 3).reshape(BATCH, HIST, DIM)


# speedup vs baseline: 1.5988x; 1.5605x over previous
import functools

import jax
import jax.numpy as jnp
from jax import lax
from jax.experimental import pallas as pl
from jax.experimental.pallas import tpu as pltpu
from jax.experimental.pallas import tpu_sc as plsc

VOCAB = 1000000
DIM = 64
BATCH = 16384
HIST = 50

_NW = 32
_BPW = BATCH // _NW
_CB = 8
_NCHUNK = _BPW // _CB
_HALF = _BPW // 2
_HP = 56


def _body(xt_hbm, table_hbm, out_hbm, idx_ht, idx_bv, buf0, buf1, gsem0, gsem1):
    wid = lax.axis_index("s") * 2 + lax.axis_index("c")
    b0 = wid * _BPW

    iota = lax.iota(jnp.int32, 16)
    for half in range(2):
        pltpu.sync_copy(xt_hbm.at[:, pl.ds(b0 + half * _HALF, _HALF)], idx_ht)
        for bbg in range(_HALF // 16):
            base = (half * _HALF + bbg * 16) * _HP + iota * _HP

            @pl.loop(0, HIST)
            def _(h):
                v = idx_ht[h, pl.ds(bbg * 16, 16)]
                plsc.store_scatter(idx_bv, [base + h], v)

    bufs = (buf0, buf1)
    gsems = (gsem0, gsem1)

    def fire(c, b):
        for k in range(_CB):
            off = pl.multiple_of((c * _CB + k) * _HP, 8)
            pltpu.async_copy(
                table_hbm.at[idx_bv.at[pl.ds(off, HIST)]],
                bufs[b].at[k],
                gsems[b],
            )

    def drain(b):
        for k in range(_CB):
            pltpu.make_async_copy(
                table_hbm.at[idx_bv.at[pl.ds(k * _HP, HIST)]],
                bufs[b].at[k],
                gsems[b],
            ).wait()

    fire(0, 0)

    @pl.loop(0, _NCHUNK, step=2)
    def _(c):
        for b in range(2):
            cc = c + b

            @pl.when(cc + 1 < _NCHUNK)
            def _():
                fire(cc + 1, 1 - b)

            drain(b)
            pltpu.sync_copy(bufs[b], out_hbm.at[pl.ds(b0 + cc * _CB, _CB)])


@jax.jit
def kernel(x, table):
    mesh = plsc.VectorSubcoreMesh(core_axis_name="c", subcore_axis_name="s")
    out = pl.kernel(
        _body,
        out_type=jax.ShapeDtypeStruct((BATCH, HIST, DIM), jnp.float32),
        mesh=mesh,
        scratch_types=[
            pltpu.VMEM((HIST, _HALF), jnp.int32),
            pltpu.VMEM((_BPW * _HP,), jnp.int32),
            pltpu.VMEM((_CB, HIST, DIM), jnp.float32),
            pltpu.VMEM((_CB, HIST, DIM), jnp.float32),
            pltpu.SemaphoreType.DMA,
            pltpu.SemaphoreType.DMA,
        ],
        compiler_params=pltpu.CompilerParams(
            use_tc_tiling_on_sc=False, needs_layout_passes=False
        ),
    )(x.T.astype(jnp.int32), table)
    return out
